# baseline (speedup 1.0000x reference)
import jax
import jax.numpy as jnp
from jax import lax
from jax.experimental import pallas as pl
from jax.experimental.pallas import tpu as pltpu
from jax.experimental.pallas import tpu_sc as plsc

_VOCAB = 1000
_EMBED_DIM = 128
_BATCH = 4096
_HIST_LEN = 200
_NUM_IDX = _BATCH * _HIST_LEN
_NUM_WORKERS = 32
_PER_W = _NUM_IDX // _NUM_WORKERS
_ROUND = 256
_NA = 192
_NB = _ROUND - _NA
_NROUNDS = _PER_W // _ROUND


def kernel(input_ids, table):
    idx = input_ids.reshape(_NUM_IDX).astype(jnp.int32)

    mesh = plsc.VectorSubcoreMesh(
        core_axis_name="core", subcore_axis_name="subcore"
    )

    @pl.kernel(
        out_type=jax.ShapeDtypeStruct((_NUM_IDX, _EMBED_DIM), table.dtype),
        mesh=mesh,
        scratch_types=[
            pltpu.VMEM_SHARED((_VOCAB, _EMBED_DIM), jnp.float32),
            pltpu.VMEM((_PER_W,), jnp.int32),
            pltpu.VMEM((2, _NA, _EMBED_DIM), jnp.float32),
            pltpu.VMEM((2, _NB, _EMBED_DIM), jnp.float32),
            pltpu.SemaphoreType.DMA,
            pltpu.SemaphoreType.DMA,
            pltpu.SemaphoreType.DMA,
            pltpu.SemaphoreType.DMA,
            pltpu.SemaphoreType.DMA,
        ],
    )
    def sc_gather(
        table_hbm, idx_hbm, out_hbm, table_sh, idx_v, buf_a, buf_b,
        sem_i, sem_ga, sem_gb, sem_oa, sem_ob,
    ):
        cid = lax.axis_index("core")
        sid = lax.axis_index("subcore")
        wid = sid * 2 + cid
        base = wid * _PER_W

        pltpu.make_async_copy(
            idx_hbm.at[pl.ds(base, _PER_W)], idx_v, sem_i
        ).start()

        @pl.when(sid < 5)
        def _():
            rows = _VOCAB // 5
            pltpu.async_copy(
                table_hbm.at[pl.ds(sid * rows, rows)],
                table_sh.at[pl.ds(sid * rows, rows)],
                sem_ga,
            ).wait()

        pltpu.make_async_copy(
            idx_hbm.at[pl.ds(base, _PER_W)], idx_v, sem_i
        ).wait()
        plsc.subcore_barrier()

        def start_gathers(r):
            buf = lax.rem(r, 2)
            off = r * _ROUND
            pltpu.make_async_copy(
                table_sh.at[idx_v.at[pl.ds(off, _NA)]],
                buf_a.at[buf],
                sem_ga,
            ).start()
            pltpu.make_async_copy(
                table_hbm.at[idx_v.at[pl.ds(off + _NA, _NB)]],
                buf_b.at[buf],
                sem_gb,
            ).start()

        def wait_gathers_start_out(r):
            buf = lax.rem(r, 2)
            hoff = base + r * _ROUND
            pltpu.make_async_copy(
                table_sh.at[idx_v.at[pl.ds(0, _NA)]], buf_a.at[buf], sem_ga
            ).wait()
            pltpu.make_async_copy(
                buf_a.at[buf], out_hbm.at[pl.ds(hoff, _NA)], sem_oa
            ).start()
            pltpu.make_async_copy(
                table_hbm.at[idx_v.at[pl.ds(0, _NB)]], buf_b.at[buf], sem_gb
            ).wait()
            pltpu.make_async_copy(
                buf_b.at[buf], out_hbm.at[pl.ds(hoff + _NA, _NB)], sem_ob
            ).start()

        def wait_outs(r):
            buf = lax.rem(r, 2)
            pltpu.make_async_copy(
                buf_a.at[buf], out_hbm.at[pl.ds(base, _NA)], sem_oa
            ).wait()
            pltpu.make_async_copy(
                buf_b.at[buf], out_hbm.at[pl.ds(base, _NB)], sem_ob
            ).wait()

        start_gathers(0)

        @pl.loop(1, _NROUNDS + 1)
        def _(r):
            @pl.when(r >= 2)
            def _():
                wait_outs(r - 2)

            @pl.when(r < _NROUNDS)
            def _():
                start_gathers(r)

            wait_gathers_start_out(r - 1)

        wait_outs(_NROUNDS - 1)

    out = sc_gather(table, idx)
    return out.reshape(_BATCH, _HIST_LEN, _EMBED_DIM)


# speedup vs baseline: 1.2617x; 1.2617x over previous
import jax
import jax.numpy as jnp
from jax import lax
from jax.experimental import pallas as pl
from jax.experimental.pallas import tpu as pltpu
from jax.experimental.pallas import tpu_sc as plsc

_VOCAB = 1000
_EMBED_DIM = 128
_BATCH = 4096
_HIST_LEN = 200
_NUM_IDX = _BATCH * _HIST_LEN
_WINDOW = 256


def kernel(input_ids, table):
    idx = input_ids.reshape(1, _NUM_IDX).astype(jnp.int32)

    mesh = plsc.VectorSubcoreMesh(
        core_axis_name="core", subcore_axis_name="subcore"
    )

    @pl.kernel(
        out_type=jax.ShapeDtypeStruct((_NUM_IDX, _EMBED_DIM), table.dtype),
        mesh=mesh,
        scratch_types=[
            pltpu.VMEM_SHARED((_VOCAB, _EMBED_DIM), jnp.float32),
            pltpu.SemaphoreType.DMA,
        ],
    )
    def sc_gather(table_hbm, idx_hbm, out_hbm, table_sh, sem):
        sid = lax.axis_index("subcore")

        @pl.when(sid < 5)
        def _():
            rows = _VOCAB // 5
            pltpu.async_copy(
                table_hbm.at[pl.ds(sid * rows, rows)],
                table_sh.at[pl.ds(sid * rows, rows)],
                sem,
            ).wait()

        plsc.subcore_barrier()

        def body(i_vmem, o_vmem):
            pltpu.sync_copy(table_sh.at[i_vmem.at[0]], o_vmem)

        pltpu.emit_pipeline(
            body,
            grid=(_NUM_IDX // _WINDOW,),
            in_specs=[
                pl.BlockSpec((1, _WINDOW), index_map=lambda i: (0, i))
            ],
            out_specs=[
                pl.BlockSpec((_WINDOW, _EMBED_DIM), index_map=lambda i: (i, 0))
            ],
            core_axis_name=("core", "subcore"),
            dimension_semantics=(pltpu.PARALLEL,),
        )(idx_hbm, out_hbm)

    out = sc_gather(table, idx)
    return out.reshape(_BATCH, _HIST_LEN, _EMBED_DIM)
